# baseline (speedup 1.0000x reference)
import functools

import jax
import jax.numpy as jnp
from jax import lax
from jax.experimental import pallas as pl
from jax.experimental.pallas import tpu as pltpu
from jax.experimental.pallas import tpu_sc as plsc

NUM_SAMPLES = 16


def _sc_gather(ids, adjT, centrality):
  B = ids.shape[0]
  D, N = adjT.shape
  info = plsc.get_sparse_core_info()
  NC, NS, L = info.num_cores, info.num_subcores, info.num_lanes
  NW = NC * NS
  assert D == NW and B % (8 * L) == 0
  CB = 4096
  assert B % CB == 0

  mesh = plsc.VectorSubcoreMesh(core_axis_name="c", subcore_axis_name="s")

  @functools.partial(
      pl.kernel,
      out_type=[
          jax.ShapeDtypeStruct((D, B), jnp.int32),
          jax.ShapeDtypeStruct((D, B), jnp.float32),
      ],
      mesh=mesh,
      compiler_params=pltpu.CompilerParams(
          needs_layout_passes=False, use_tc_tiling_on_sc=True),
      scratch_types=[
          pltpu.VMEM((CB,), jnp.int32),
          pltpu.VMEM((CB,), jnp.int32),
          pltpu.VMEM((CB,), jnp.int32),
          pltpu.VMEM((CB,), jnp.int32),
          pltpu.VMEM((CB,), jnp.float32),
          pltpu.VMEM((CB,), jnp.float32),
          pltpu.VMEM((N,), jnp.int32),
          pltpu.VMEM((N,), jnp.float32),
          pltpu.VMEM_SHARED((N,), jnp.float32),
          pltpu.VMEM_SHARED((B,), jnp.int32),
          pltpu.SemaphoreType.DMA,
          pltpu.SemaphoreType.DMA,
          pltpu.SemaphoreType.DMA,
          pltpu.SemaphoreType.DMA,
      ],
  )
  def k(ids_h, adjT_h, cent_h, adjT_o, scoT_o,
        ida_v, idb_v, nba_v, nbb_v, sca_v, scb_v, row_v, cent_v, cent_sh,
        ids_sh, sem_i, sem_r, sem_c, sem_o):
    sid = lax.axis_index("s")
    d = sid * NC + lax.axis_index("c")
    row_cp = pltpu.async_copy(adjT_h.at[d], row_v, sem_r)
    SL = 3200
    off = jnp.where(sid == NS - 1, N - SL, sid * SL)
    pltpu.sync_copy(cent_h.at[pl.ds(off, SL)], cent_v.at[pl.ds(off, SL)])
    pltpu.sync_copy(cent_v.at[pl.ds(off, SL)], cent_sh.at[pl.ds(off, SL)])
    ISL = B // NS
    ioff = sid * ISL
    pltpu.sync_copy(ids_h.at[pl.ds(ioff, ISL)], ida_v.at[pl.ds(0, ISL)])
    pltpu.sync_copy(ida_v.at[pl.ds(0, ISL)], ids_sh.at[pl.ds(ioff, ISL)])
    plsc.subcore_barrier()
    cent_cp = pltpu.async_copy(cent_sh, cent_v, sem_c)
    pltpu.sync_copy(ids_sh.at[pl.ds(0, CB)], ida_v)
    row_cp.wait()
    cent_cp.wait()

    nchunks = B // CB
    ibufs = (ida_v, idb_v)
    obufs = ((nba_v, sca_v), (nbb_v, scb_v))

    for ch in range(nchunks):
      cur = ibufs[ch % 2]
      nbr_v, sco_v = obufs[ch % 2]
      if ch + 1 < nchunks:
        nxt_cp = pltpu.async_copy(
            ids_sh.at[pl.ds((ch + 1) * CB, CB)], ibufs[(ch + 1) % 2], sem_i)
      if ch >= 2:
        pltpu.make_async_copy(nbr_v, adjT_o.at[d, pl.ds((ch - 2) * CB, CB)],
                              sem_o).wait()
        pltpu.make_async_copy(sco_v, scoT_o.at[d, pl.ds((ch - 2) * CB, CB)],
                              sem_o).wait()

      def body(t, _, cur=cur, nbr_v=nbr_v, sco_v=sco_v):
        idvec = cur[pl.ds(t * L, L)]
        nbr = plsc.load_gather(row_v, [idvec])
        nbr_v[pl.ds(t * L, L)] = nbr
        sco_v[pl.ds(t * L, L)] = plsc.load_gather(cent_v, [nbr])
        return 0

      lax.fori_loop(0, CB // L, body, 0, unroll=16)

      pltpu.async_copy(nbr_v, adjT_o.at[d, pl.ds(ch * CB, CB)], sem_o)
      pltpu.async_copy(sco_v, scoT_o.at[d, pl.ds(ch * CB, CB)], sem_o)
      if ch + 1 < nchunks:
        nxt_cp.wait()

    for ch in (nchunks - 2, nchunks - 1):
      nbr_v, sco_v = obufs[ch % 2]
      pltpu.make_async_copy(nbr_v, adjT_o.at[d, pl.ds(ch * CB, CB)],
                            sem_o).wait()
      pltpu.make_async_copy(sco_v, scoT_o.at[d, pl.ds(ch * CB, CB)],
                            sem_o).wait()

  return k(ids, adjT, centrality)


def _tc_select(adjT, scoT):
  D, B = adjT.shape
  BW = 4096
  assert B % BW == 0

  def body(adj_ref, sco_ref, out_ref):
    a = adj_ref[...]
    s2 = lax.bitcast_convert_type(sco_ref[...], jnp.int32) * 2
    row = lax.broadcasted_iota(jnp.int32, (D, 1), 0)
    rank = jnp.zeros((D, BW), jnp.int32)
    for j in range(D):
      tie = (row > j).astype(jnp.int32)
      kj = s2[j:j + 1, :] + tie
      rank = rank + (kj > s2).astype(jnp.int32)
    for p in range(NUM_SAMPLES):
      sel = jnp.where(rank == p, a, 0)
      out_ref[p:p + 1, :] = jnp.sum(sel, axis=0, keepdims=True)

  return pl.pallas_call(
      body,
      grid=(B // BW,),
      in_specs=[
          pl.BlockSpec((D, BW), lambda g: (0, g)),
          pl.BlockSpec((D, BW), lambda g: (0, g)),
      ],
      out_specs=pl.BlockSpec((NUM_SAMPLES, BW), lambda g: (0, g)),
      out_shape=jax.ShapeDtypeStruct((NUM_SAMPLES, B), jnp.int32),
  )(adjT, scoT)


def kernel(ids, num_samples, adj_info, centrality):
  del num_samples
  adjT, scoT = _sc_gather(ids, adj_info.T, centrality)
  outT = _tc_select(adjT, scoT)
  return outT.T


# speedup vs baseline: 1.2750x; 1.2750x over previous
import functools

import jax
import jax.numpy as jnp
from jax import lax
from jax.experimental import pallas as pl
from jax.experimental.pallas import tpu as pltpu
from jax.experimental.pallas import tpu_sc as plsc

NUM_SAMPLES = 16


def _sc_gather(ids, adjT, centrality):
  B = ids.shape[0]
  D, N = adjT.shape
  info = plsc.get_sparse_core_info()
  NC, NS, L = info.num_cores, info.num_subcores, info.num_lanes
  NW = NC * NS
  assert D == NW and B % (8 * L) == 0
  CB = 4096
  assert B % CB == 0

  mesh = plsc.VectorSubcoreMesh(core_axis_name="c", subcore_axis_name="s")

  @functools.partial(
      pl.kernel,
      out_type=[
          jax.ShapeDtypeStruct((D, B), jnp.int32),
          jax.ShapeDtypeStruct((D, B), jnp.float32),
      ],
      mesh=mesh,
      compiler_params=pltpu.CompilerParams(
          needs_layout_passes=False, use_tc_tiling_on_sc=True),
      scratch_types=[
          pltpu.VMEM((CB,), jnp.int32),
          pltpu.VMEM((CB,), jnp.int32),
          pltpu.VMEM((CB,), jnp.int32),
          pltpu.VMEM((CB,), jnp.int32),
          pltpu.VMEM((CB,), jnp.float32),
          pltpu.VMEM((CB,), jnp.float32),
          pltpu.VMEM((N,), jnp.int32),
          pltpu.VMEM((N,), jnp.float32),
          pltpu.VMEM_SHARED((N,), jnp.float32),
          pltpu.VMEM_SHARED((B,), jnp.int32),
          pltpu.SemaphoreType.DMA,
          pltpu.SemaphoreType.DMA,
          pltpu.SemaphoreType.DMA,
          pltpu.SemaphoreType.DMA,
      ],
  )
  def k(ids_h, adjT_h, cent_h, adjT_o, scoT_o,
        ida_v, idb_v, nba_v, nbb_v, sca_v, scb_v, row_v, cent_v, cent_sh,
        ids_sh, sem_i, sem_r, sem_c, sem_o):
    sid = lax.axis_index("s")
    d = sid * NC + lax.axis_index("c")
    row_cp = pltpu.async_copy(adjT_h.at[d], row_v, sem_r)
    SL = 3200
    off = jnp.where(sid == NS - 1, N - SL, sid * SL)
    pltpu.sync_copy(cent_h.at[pl.ds(off, SL)], cent_v.at[pl.ds(off, SL)])
    pltpu.sync_copy(cent_v.at[pl.ds(off, SL)], cent_sh.at[pl.ds(off, SL)])
    ISL = B // NS
    ioff = sid * ISL
    pltpu.sync_copy(ids_h.at[pl.ds(ioff, ISL)], ida_v.at[pl.ds(0, ISL)])
    pltpu.sync_copy(ida_v.at[pl.ds(0, ISL)], ids_sh.at[pl.ds(ioff, ISL)])
    plsc.subcore_barrier()
    cent_cp = pltpu.async_copy(cent_sh, cent_v, sem_c)
    pltpu.sync_copy(ids_sh.at[pl.ds(0, CB)], ida_v)
    row_cp.wait()
    cent_cp.wait()

    nchunks = B // CB
    ibufs = (ida_v, idb_v)
    obufs = ((nba_v, sca_v), (nbb_v, scb_v))

    for ch in range(nchunks):
      cur = ibufs[ch % 2]
      nbr_v, sco_v = obufs[ch % 2]
      if ch + 1 < nchunks:
        nxt_cp = pltpu.async_copy(
            ids_sh.at[pl.ds((ch + 1) * CB, CB)], ibufs[(ch + 1) % 2], sem_i)
      if ch >= 2:
        pltpu.make_async_copy(nbr_v, adjT_o.at[d, pl.ds((ch - 2) * CB, CB)],
                              sem_o).wait()
        pltpu.make_async_copy(sco_v, scoT_o.at[d, pl.ds((ch - 2) * CB, CB)],
                              sem_o).wait()

      @plsc.parallel_loop(0, CB, L, unroll=16)
      def _(t, cur=cur, nbr_v=nbr_v, sco_v=sco_v):
        idvec = cur[pl.ds(t, L)]
        nbr = plsc.load_gather(row_v, [idvec])
        nbr_v[pl.ds(t, L)] = nbr
        sco_v[pl.ds(t, L)] = plsc.load_gather(cent_v, [nbr])

      pltpu.async_copy(nbr_v, adjT_o.at[d, pl.ds(ch * CB, CB)], sem_o)
      pltpu.async_copy(sco_v, scoT_o.at[d, pl.ds(ch * CB, CB)], sem_o)
      if ch + 1 < nchunks:
        nxt_cp.wait()

    for ch in (nchunks - 2, nchunks - 1):
      nbr_v, sco_v = obufs[ch % 2]
      pltpu.make_async_copy(nbr_v, adjT_o.at[d, pl.ds(ch * CB, CB)],
                            sem_o).wait()
      pltpu.make_async_copy(sco_v, scoT_o.at[d, pl.ds(ch * CB, CB)],
                            sem_o).wait()

  return k(ids, adjT, centrality)


def _tc_select(adjT, scoT):
  D, B = adjT.shape
  BW = 4096
  assert B % BW == 0

  def body(adj_ref, sco_ref, out_ref):
    a = adj_ref[...]
    s2 = lax.bitcast_convert_type(sco_ref[...], jnp.int32) * 2
    row = lax.broadcasted_iota(jnp.int32, (D, 1), 0)
    rank = jnp.zeros((D, BW), jnp.int32)
    for j in range(D):
      tie = (row > j).astype(jnp.int32)
      kj = s2[j:j + 1, :] + tie
      rank = rank + (kj > s2).astype(jnp.int32)
    for p in range(NUM_SAMPLES):
      sel = jnp.where(rank == p, a, 0)
      out_ref[p:p + 1, :] = jnp.sum(sel, axis=0, keepdims=True)

  return pl.pallas_call(
      body,
      grid=(B // BW,),
      in_specs=[
          pl.BlockSpec((D, BW), lambda g: (0, g)),
          pl.BlockSpec((D, BW), lambda g: (0, g)),
      ],
      out_specs=pl.BlockSpec((NUM_SAMPLES, BW), lambda g: (0, g)),
      out_shape=jax.ShapeDtypeStruct((NUM_SAMPLES, B), jnp.int32),
  )(adjT, scoT)


def kernel(ids, num_samples, adj_info, centrality):
  del num_samples
  adjT, scoT = _sc_gather(ids, adj_info.T, centrality)
  outT = _tc_select(adjT, scoT)
  return outT.T


# speedup vs baseline: 1.2840x; 1.0071x over previous
import functools

import jax
import jax.numpy as jnp
from jax import lax
from jax.experimental import pallas as pl
from jax.experimental.pallas import tpu as pltpu
from jax.experimental.pallas import tpu_sc as plsc

NUM_SAMPLES = 16


def _sc_gather(ids, adjT, centrality):
  B = ids.shape[0]
  D, N = adjT.shape
  info = plsc.get_sparse_core_info()
  NC, NS, L = info.num_cores, info.num_subcores, info.num_lanes
  NW = NC * NS
  assert D == NW and B % (8 * L) == 0
  CB = 4096
  assert B % CB == 0

  mesh = plsc.VectorSubcoreMesh(core_axis_name="c", subcore_axis_name="s")

  @functools.partial(
      pl.kernel,
      out_type=[
          jax.ShapeDtypeStruct((D, B), jnp.int32),
          jax.ShapeDtypeStruct((D, B), jnp.float32),
      ],
      mesh=mesh,
      compiler_params=pltpu.CompilerParams(
          needs_layout_passes=False, use_tc_tiling_on_sc=True),
      scratch_types=[
          pltpu.VMEM((CB,), jnp.int32),
          pltpu.VMEM((CB,), jnp.int32),
          pltpu.VMEM((CB,), jnp.int32),
          pltpu.VMEM((CB,), jnp.int32),
          pltpu.VMEM((CB,), jnp.float32),
          pltpu.VMEM((CB,), jnp.float32),
          pltpu.VMEM((N,), jnp.int32),
          pltpu.VMEM((N,), jnp.float32),
          pltpu.VMEM_SHARED((N,), jnp.float32),
          pltpu.VMEM_SHARED((B,), jnp.int32),
          pltpu.SemaphoreType.DMA,
          pltpu.SemaphoreType.DMA,
          pltpu.SemaphoreType.DMA,
          pltpu.SemaphoreType.DMA,
      ],
  )
  def k(ids_h, adjT_h, cent_h, adjT_o, scoT_o,
        ida_v, idb_v, nba_v, nbb_v, sca_v, scb_v, row_v, cent_v, cent_sh,
        ids_sh, sem_i, sem_r, sem_c, sem_o):
    sid = lax.axis_index("s")
    d = sid * NC + lax.axis_index("c")
    row_cp = pltpu.async_copy(adjT_h.at[d], row_v, sem_r)
    SL = 3200
    off = jnp.where(sid == NS - 1, N - SL, sid * SL)
    pltpu.sync_copy(cent_h.at[pl.ds(off, SL)], cent_v.at[pl.ds(off, SL)])
    pltpu.sync_copy(cent_v.at[pl.ds(off, SL)], cent_sh.at[pl.ds(off, SL)])
    ISL = B // NS
    ioff = sid * ISL
    pltpu.sync_copy(ids_h.at[pl.ds(ioff, ISL)], ida_v.at[pl.ds(0, ISL)])
    pltpu.sync_copy(ida_v.at[pl.ds(0, ISL)], ids_sh.at[pl.ds(ioff, ISL)])
    plsc.subcore_barrier()
    cent_cp = pltpu.async_copy(cent_sh, cent_v, sem_c)
    pltpu.sync_copy(ids_sh.at[pl.ds(0, CB)], ida_v)
    row_cp.wait()
    cent_cp.wait()

    nchunks = B // CB
    ibufs = (ida_v, idb_v)
    obufs = ((nba_v, sca_v), (nbb_v, scb_v))

    for ch in range(nchunks):
      cur = ibufs[ch % 2]
      nbr_v, sco_v = obufs[ch % 2]
      if ch + 1 < nchunks:
        nxt_cp = pltpu.async_copy(
            ids_sh.at[pl.ds((ch + 1) * CB, CB)], ibufs[(ch + 1) % 2], sem_i)
      if ch >= 2:
        pltpu.make_async_copy(nbr_v, adjT_o.at[d, pl.ds((ch - 2) * CB, CB)],
                              sem_o).wait()
        pltpu.make_async_copy(sco_v, scoT_o.at[d, pl.ds((ch - 2) * CB, CB)],
                              sem_o).wait()

      @plsc.parallel_loop(0, CB, L, unroll=8)
      def _(t, cur=cur, nbr_v=nbr_v, sco_v=sco_v):
        idvec = cur[pl.ds(t, L)]
        nbr = plsc.load_gather(row_v, [idvec])
        nbr_v[pl.ds(t, L)] = nbr
        sco_v[pl.ds(t, L)] = plsc.load_gather(cent_v, [nbr])

      pltpu.async_copy(nbr_v, adjT_o.at[d, pl.ds(ch * CB, CB)], sem_o)
      pltpu.async_copy(sco_v, scoT_o.at[d, pl.ds(ch * CB, CB)], sem_o)
      if ch + 1 < nchunks:
        nxt_cp.wait()

    for ch in (nchunks - 2, nchunks - 1):
      nbr_v, sco_v = obufs[ch % 2]
      pltpu.make_async_copy(nbr_v, adjT_o.at[d, pl.ds(ch * CB, CB)],
                            sem_o).wait()
      pltpu.make_async_copy(sco_v, scoT_o.at[d, pl.ds(ch * CB, CB)],
                            sem_o).wait()

  return k(ids, adjT, centrality)


def _tc_select(adjT, scoT):
  D, B = adjT.shape
  BW = 4096
  assert B % BW == 0

  def body(adj_ref, sco_ref, out_ref):
    a = adj_ref[...]
    s2 = lax.bitcast_convert_type(sco_ref[...], jnp.int32) * 2
    row = lax.broadcasted_iota(jnp.int32, (D, 1), 0)
    rank = jnp.zeros((D, BW), jnp.int32)
    for j in range(D):
      tie = (row > j).astype(jnp.int32)
      kj = s2[j:j + 1, :] + tie
      rank = rank + (kj > s2).astype(jnp.int32)
    for p in range(NUM_SAMPLES):
      sel = jnp.where(rank == p, a, 0)
      out_ref[p:p + 1, :] = jnp.sum(sel, axis=0, keepdims=True)

  return pl.pallas_call(
      body,
      grid=(B // BW,),
      in_specs=[
          pl.BlockSpec((D, BW), lambda g: (0, g)),
          pl.BlockSpec((D, BW), lambda g: (0, g)),
      ],
      out_specs=pl.BlockSpec((NUM_SAMPLES, BW), lambda g: (0, g)),
      out_shape=jax.ShapeDtypeStruct((NUM_SAMPLES, B), jnp.int32),
  )(adjT, scoT)


def kernel(ids, num_samples, adj_info, centrality):
  del num_samples
  adjT, scoT = _sc_gather(ids, adj_info.T, centrality)
  outT = _tc_select(adjT, scoT)
  return outT.T


# speedup vs baseline: 1.2858x; 1.0014x over previous
import functools

import jax
import jax.numpy as jnp
from jax import lax
from jax.experimental import pallas as pl
from jax.experimental.pallas import tpu as pltpu
from jax.experimental.pallas import tpu_sc as plsc

NUM_SAMPLES = 16


def _sc_gather(ids, adjT, centrality):
  B = ids.shape[0]
  D, N = adjT.shape
  info = plsc.get_sparse_core_info()
  NC, NS, L = info.num_cores, info.num_subcores, info.num_lanes
  NW = NC * NS
  assert D == NW and B % (8 * L) == 0
  CB = 4096
  assert B % CB == 0

  mesh = plsc.VectorSubcoreMesh(core_axis_name="c", subcore_axis_name="s")

  @functools.partial(
      pl.kernel,
      out_type=[
          jax.ShapeDtypeStruct((D, B), jnp.int32),
          jax.ShapeDtypeStruct((D, B), jnp.float32),
      ],
      mesh=mesh,
      compiler_params=pltpu.CompilerParams(
          needs_layout_passes=False, use_tc_tiling_on_sc=True),
      scratch_types=[
          pltpu.VMEM((CB,), jnp.int32),
          pltpu.VMEM((CB,), jnp.int32),
          pltpu.VMEM((CB,), jnp.int32),
          pltpu.VMEM((CB,), jnp.int32),
          pltpu.VMEM((CB,), jnp.float32),
          pltpu.VMEM((CB,), jnp.float32),
          pltpu.VMEM((N,), jnp.int32),
          pltpu.VMEM((N,), jnp.float32),
          pltpu.VMEM_SHARED((N,), jnp.float32),
          pltpu.VMEM_SHARED((B,), jnp.int32),
          pltpu.SemaphoreType.DMA,
          pltpu.SemaphoreType.DMA,
          pltpu.SemaphoreType.DMA,
          pltpu.SemaphoreType.DMA,
      ],
  )
  def k(ids_h, adjT_h, cent_h, adjT_o, scoT_o,
        ida_v, idb_v, nba_v, nbb_v, sca_v, scb_v, row_v, cent_v, cent_sh,
        ids_sh, sem_i, sem_r, sem_c, sem_o):
    sid = lax.axis_index("s")
    d = sid * NC + lax.axis_index("c")
    row_cp = pltpu.async_copy(adjT_h.at[d], row_v, sem_r)
    SL = 3200
    off = jnp.where(sid == NS - 1, N - SL, sid * SL)
    pltpu.sync_copy(cent_h.at[pl.ds(off, SL)], cent_v.at[pl.ds(off, SL)])
    pltpu.sync_copy(cent_v.at[pl.ds(off, SL)], cent_sh.at[pl.ds(off, SL)])
    ISL = B // NS
    ioff = sid * ISL
    pltpu.sync_copy(ids_h.at[pl.ds(ioff, ISL)], ida_v.at[pl.ds(0, ISL)])
    pltpu.sync_copy(ida_v.at[pl.ds(0, ISL)], ids_sh.at[pl.ds(ioff, ISL)])
    plsc.subcore_barrier()
    cent_cp = pltpu.async_copy(cent_sh, cent_v, sem_c)
    pltpu.sync_copy(ids_sh.at[pl.ds(0, CB)], ida_v)
    row_cp.wait()
    cent_cp.wait()

    nchunks = B // CB
    ibufs = (ida_v, idb_v)
    obufs = ((nba_v, sca_v), (nbb_v, scb_v))

    for ch in range(nchunks):
      cur = ibufs[ch % 2]
      nbr_v, sco_v = obufs[ch % 2]
      if ch + 1 < nchunks:
        nxt_cp = pltpu.async_copy(
            ids_sh.at[pl.ds((ch + 1) * CB, CB)], ibufs[(ch + 1) % 2], sem_i)
      if ch >= 2:
        pltpu.make_async_copy(nbr_v, adjT_o.at[d, pl.ds((ch - 2) * CB, CB)],
                              sem_o).wait()
        pltpu.make_async_copy(sco_v, scoT_o.at[d, pl.ds((ch - 2) * CB, CB)],
                              sem_o).wait()

      @plsc.parallel_loop(0, CB, L, unroll=4)
      def _(t, cur=cur, nbr_v=nbr_v, sco_v=sco_v):
        idvec = cur[pl.ds(t, L)]
        nbr = plsc.load_gather(row_v, [idvec])
        nbr_v[pl.ds(t, L)] = nbr
        sco_v[pl.ds(t, L)] = plsc.load_gather(cent_v, [nbr])

      pltpu.async_copy(nbr_v, adjT_o.at[d, pl.ds(ch * CB, CB)], sem_o)
      pltpu.async_copy(sco_v, scoT_o.at[d, pl.ds(ch * CB, CB)], sem_o)
      if ch + 1 < nchunks:
        nxt_cp.wait()

    for ch in (nchunks - 2, nchunks - 1):
      nbr_v, sco_v = obufs[ch % 2]
      pltpu.make_async_copy(nbr_v, adjT_o.at[d, pl.ds(ch * CB, CB)],
                            sem_o).wait()
      pltpu.make_async_copy(sco_v, scoT_o.at[d, pl.ds(ch * CB, CB)],
                            sem_o).wait()

  return k(ids, adjT, centrality)


def _tc_select(adjT, scoT):
  D, B = adjT.shape
  BW = 4096
  assert B % BW == 0

  def body(adj_ref, sco_ref, out_ref):
    a = adj_ref[...]
    s2 = lax.bitcast_convert_type(sco_ref[...], jnp.int32) * 2
    row = lax.broadcasted_iota(jnp.int32, (D, 1), 0)
    rank = jnp.zeros((D, BW), jnp.int32)
    for j in range(D):
      tie = (row > j).astype(jnp.int32)
      kj = s2[j:j + 1, :] + tie
      rank = rank + (kj > s2).astype(jnp.int32)
    for p in range(NUM_SAMPLES):
      sel = jnp.where(rank == p, a, 0)
      out_ref[p:p + 1, :] = jnp.sum(sel, axis=0, keepdims=True)

  return pl.pallas_call(
      body,
      grid=(B // BW,),
      in_specs=[
          pl.BlockSpec((D, BW), lambda g: (0, g)),
          pl.BlockSpec((D, BW), lambda g: (0, g)),
      ],
      out_specs=pl.BlockSpec((NUM_SAMPLES, BW), lambda g: (0, g)),
      out_shape=jax.ShapeDtypeStruct((NUM_SAMPLES, B), jnp.int32),
  )(adjT, scoT)


def kernel(ids, num_samples, adj_info, centrality):
  del num_samples
  adjT, scoT = _sc_gather(ids, adj_info.T, centrality)
  outT = _tc_select(adjT, scoT)
  return outT.T
